# Initial kernel scaffold; baseline (speedup 1.0000x reference)
import jax
import jax.numpy as jnp
from jax.experimental import pallas as pl


def kernel(x, concepts, Wq, bq, Wk, bk, Wv, bv, Wo, bo, keW1, keb1, keg1, kebe1, keW2, keb2, selW, selb, skW1, skb1, skW2, skb2, fuW1, fub1, fug, fube, fuW2, fub2):
    raise NotImplementedError("write your pallas kernel here")


# speedup vs baseline: 1.0700x; 1.0700x over previous
import functools
import math

import jax
import jax.numpy as jnp
from jax.experimental import pallas as pl
from jax.experimental.pallas import tpu as pltpu

DIM = 768
H = 8
HD = DIM // H
NC = 1000
NS = 50
F32 = jnp.float32


def _body(x_ref, concepts_ref, wq_ref, bq_ref, wk_ref, wv_ref, bv_ref,
          wo_ref, bo_ref, keW1_ref, keb1_ref, keg1_ref, kebe1_ref,
          keW2_ref, keb2_ref, selW_ref, selb_ref,
          skW1_ref, skb1_ref, skW2_ref, skb2_ref,
          fuW1l_ref, fub1_ref, fug_ref, fube_ref, fuW2_ref, fub2_ref,
          out_ref, sem_ref, acc_ref, sc_ref):
    k = pl.program_id(0)
    x = x_ref[...]

    @pl.when(k == 0)
    def _init():
        sl = jnp.dot(x, selW_ref[...], preferred_element_type=F32) + selb_ref[...]
        sl = sl - jnp.max(sl, axis=-1, keepdims=True)
        e = jnp.exp(sl)
        sc_ref[...] = e / jnp.sum(e, axis=-1, keepdims=True)

        q = jnp.dot(x, wq_ref[...], preferred_element_type=F32) + bq_ref[...]
        rows = jax.lax.broadcasted_iota(jnp.int32, (H, DIM), 0)
        cols = jax.lax.broadcasted_iota(jnp.int32, (H, DIM), 1)
        maskf = (cols // HD == rows).astype(F32)
        q8 = maskf * q
        t = jax.lax.dot_general(wk_ref[...], q8, (((1,), (1,)), ((), ())),
                                preferred_element_type=F32)
        logits = jnp.dot(concepts_ref[...], t,
                         preferred_element_type=F32) * (1.0 / math.sqrt(HD))
        m = jnp.max(logits, axis=0, keepdims=True)
        ew = jnp.exp(logits - m)
        w = ew / jnp.sum(ew, axis=0, keepdims=True)
        u = jax.lax.dot_general(w, concepts_ref[...], (((0,), (0,)), ((), ())),
                                preferred_element_type=F32)
        p = jnp.dot(u, wv_ref[...], preferred_element_type=F32)
        o = jnp.sum(p * maskf, axis=0, keepdims=True) + bv_ref[...]
        attended = jnp.dot(o, wo_ref[...], preferred_element_type=F32) + bo_ref[...]
        combined = x + attended
        y = jnp.dot(combined, keW1_ref[...], preferred_element_type=F32) + keb1_ref[...]
        mu = jnp.mean(y, axis=-1, keepdims=True)
        var = jnp.mean((y - mu) * (y - mu), axis=-1, keepdims=True)
        yn = (y - mu) / jnp.sqrt(var + 1e-5) * keg1_ref[...] + kebe1_ref[...]
        h1 = jnp.maximum(yn, 0.0)
        sem_ref[...] = jnp.dot(h1, keW2_ref[...], preferred_element_type=F32) + keb2_ref[...]

    s_all = sc_ref[...]
    lane = jax.lax.broadcasted_iota(jnp.int32, (1, NS), 1)
    sk = jnp.sum(jnp.where(lane == k, s_all, 0.0))
    hk = jnp.maximum(
        jnp.dot(x, skW1_ref[0], preferred_element_type=F32) + skb1_ref[0], 0.0)
    outk = jnp.dot(hk, skW2_ref[0], preferred_element_type=F32) + skb2_ref[0]
    contrib = sk * outk

    @pl.when(k == 0)
    def _first():
        acc_ref[...] = contrib

    @pl.when(k > 0)
    def _rest():
        acc_ref[...] = acc_ref[...] + contrib

    @pl.when(k == NS - 1)
    def _fuse():
        cat = jnp.concatenate([sem_ref[...], acc_ref[...]], axis=-1)
        y = jnp.dot(cat, fuW1l_ref[...], preferred_element_type=F32) + fub1_ref[...]
        mu = jnp.mean(y, axis=-1, keepdims=True)
        var = jnp.mean((y - mu) * (y - mu), axis=-1, keepdims=True)
        yn = (y - mu) / jnp.sqrt(var + 1e-5) * fug_ref[...] + fube_ref[...]
        fh = jnp.maximum(yn, 0.0)
        out_ref[...] = jnp.dot(fh, fuW2_ref[...], preferred_element_type=F32) + fub2_ref[...]


def _const2d(shape):
    return pl.BlockSpec(shape, lambda k: (0, 0))


@jax.jit
def kernel(x, concepts, Wq, bq, Wk, bk, Wv, bv, Wo, bo, keW1, keb1, keg1,
           kebe1, keW2, keb2, selW, selb, skW1, skb1, skW2, skb2,
           fuW1, fub1, fug, fube, fuW2, fub2):
    d = DIM
    row = lambda b: b.reshape(1, -1)
    skb1r = skb1.reshape(NS, 1, d)
    skb2r = skb2.reshape(NS, 1, d)
    grid = (NS,)
    in_specs = [
        _const2d((1, d)),
        _const2d((NC, d)),
        _const2d((d, d)),
        _const2d((1, d)),
        _const2d((d, d)),
        _const2d((d, d)),
        _const2d((1, d)),
        _const2d((d, d)),
        _const2d((1, d)),
        _const2d((d, 2 * d)),
        _const2d((1, 2 * d)),
        _const2d((1, 2 * d)),
        _const2d((1, 2 * d)),
        _const2d((2 * d, d)),
        _const2d((1, d)),
        _const2d((d, NS)),
        _const2d((1, NS)),
        pl.BlockSpec((1, d, d), lambda k: (k, 0, 0)),
        pl.BlockSpec((1, 1, d), lambda k: (k, 0, 0)),
        pl.BlockSpec((1, d, d), lambda k: (k, 0, 0)),
        pl.BlockSpec((1, 1, d), lambda k: (k, 0, 0)),
        pl.BlockSpec((2 * d, 2 * d), lambda k: (1, 0)),
        _const2d((1, 2 * d)),
        _const2d((1, 2 * d)),
        _const2d((1, 2 * d)),
        _const2d((2 * d, d)),
        _const2d((1, d)),
    ]
    out = pl.pallas_call(
        _body,
        grid=grid,
        in_specs=in_specs,
        out_specs=_const2d((1, d)),
        out_shape=jax.ShapeDtypeStruct((1, d), F32),
        scratch_shapes=[
            pltpu.VMEM((1, d), F32),
            pltpu.VMEM((1, d), F32),
            pltpu.VMEM((1, NS), F32),
        ],
        compiler_params=pltpu.CompilerParams(
            dimension_semantics=("arbitrary",),
            vmem_limit_bytes=67108864,
        ),
    )(x, concepts, Wq, row(bq), Wk, Wv, row(bv), Wo, row(bo),
      keW1, row(keb1), row(keg1), row(kebe1), keW2, row(keb2),
      selW, row(selb), skW1, skb1r, skW2, skb2r,
      fuW1, row(fub1), row(fug), row(fube), fuW2, row(fub2))
    return out


# speedup vs baseline: 1.1579x; 1.0822x over previous
import functools
import math

import jax
import jax.numpy as jnp
from jax.experimental import pallas as pl
from jax.experimental.pallas import tpu as pltpu

DIM = 768
H = 8
HD = DIM // H
NC = 1000
NS = 50
SPB = 2
F32 = jnp.float32


def _body(x_ref, concepts_ref, wq_ref, bq_ref, wk_ref, wv_ref, bv_ref,
          wo_ref, bo_ref, keW1_ref, keb1_ref, keg1_ref, kebe1_ref,
          keW2_ref, keb2_ref, selW_ref, selb_ref,
          skW1_ref, skb1_ref, skW2_ref, skb2_ref,
          fuW1l_ref, fub1_ref, fug_ref, fube_ref, fuW2_ref, fub2_ref,
          out_ref, sem_ref, acc_ref, sc_ref):
    k = pl.program_id(0)
    x = x_ref[...]

    @pl.when(k == 0)
    def _init():
        sl = jnp.dot(x, selW_ref[...], preferred_element_type=F32) + selb_ref[...]
        sl = sl - jnp.max(sl, axis=-1, keepdims=True)
        e = jnp.exp(sl)
        sc_ref[...] = e / jnp.sum(e, axis=-1, keepdims=True)

        q = jnp.dot(x, wq_ref[...], preferred_element_type=F32) + bq_ref[...]
        rows = jax.lax.broadcasted_iota(jnp.int32, (H, DIM), 0)
        cols = jax.lax.broadcasted_iota(jnp.int32, (H, DIM), 1)
        maskf = (cols // HD == rows).astype(F32)
        q8 = maskf * q
        t = jax.lax.dot_general(wk_ref[...], q8, (((1,), (1,)), ((), ())),
                                preferred_element_type=F32)
        logits = jnp.dot(concepts_ref[...], t,
                         preferred_element_type=F32) * (1.0 / math.sqrt(HD))
        m = jnp.max(logits, axis=0, keepdims=True)
        ew = jnp.exp(logits - m)
        w = ew / jnp.sum(ew, axis=0, keepdims=True)
        u = jax.lax.dot_general(w, concepts_ref[...], (((0,), (0,)), ((), ())),
                                preferred_element_type=F32)
        p = jnp.dot(u, wv_ref[...], preferred_element_type=F32)
        o = jnp.sum(p * maskf, axis=0, keepdims=True) + bv_ref[...]
        attended = jnp.dot(o, wo_ref[...], preferred_element_type=F32) + bo_ref[...]
        combined = x + attended
        y = jnp.dot(combined, keW1_ref[...], preferred_element_type=F32) + keb1_ref[...]
        mu = jnp.mean(y, axis=-1, keepdims=True)
        var = jnp.mean((y - mu) * (y - mu), axis=-1, keepdims=True)
        yn = (y - mu) / jnp.sqrt(var + 1e-5) * keg1_ref[...] + kebe1_ref[...]
        h1 = jnp.maximum(yn, 0.0)
        sem_ref[...] = jnp.dot(h1, keW2_ref[...], preferred_element_type=F32) + keb2_ref[...]

    s_all = sc_ref[...]
    lane = jax.lax.broadcasted_iota(jnp.int32, (1, NS), 1)
    contrib = None
    for i in range(SPB):
        sk = jnp.sum(jnp.where(lane == SPB * k + i, s_all, 0.0))
        hk = jnp.maximum(
            jnp.dot(x, skW1_ref[i], preferred_element_type=F32) + skb1_ref[i], 0.0)
        outk = jnp.dot(hk, skW2_ref[i], preferred_element_type=F32) + skb2_ref[i]
        c = sk * outk
        contrib = c if contrib is None else contrib + c

    @pl.when(k == 0)
    def _first():
        acc_ref[...] = contrib

    @pl.when(k > 0)
    def _rest():
        acc_ref[...] = acc_ref[...] + contrib

    @pl.when(k == NS // SPB - 1)
    def _fuse():
        cat = jnp.concatenate([sem_ref[...], acc_ref[...]], axis=-1)
        y = jnp.dot(cat, fuW1l_ref[...], preferred_element_type=F32) + fub1_ref[...]
        mu = jnp.mean(y, axis=-1, keepdims=True)
        var = jnp.mean((y - mu) * (y - mu), axis=-1, keepdims=True)
        yn = (y - mu) / jnp.sqrt(var + 1e-5) * fug_ref[...] + fube_ref[...]
        fh = jnp.maximum(yn, 0.0)
        out_ref[...] = jnp.dot(fh, fuW2_ref[...], preferred_element_type=F32) + fub2_ref[...]


def _const2d(shape):
    return pl.BlockSpec(shape, lambda k: (0, 0))


@jax.jit
def kernel(x, concepts, Wq, bq, Wk, bk, Wv, bv, Wo, bo, keW1, keb1, keg1,
           kebe1, keW2, keb2, selW, selb, skW1, skb1, skW2, skb2,
           fuW1, fub1, fug, fube, fuW2, fub2):
    d = DIM
    row = lambda b: b.reshape(1, -1)
    skb1r = skb1.reshape(NS, 1, d)
    skb2r = skb2.reshape(NS, 1, d)
    grid = (NS // SPB,)
    in_specs = [
        _const2d((1, d)),
        _const2d((NC, d)),
        _const2d((d, d)),
        _const2d((1, d)),
        _const2d((d, d)),
        _const2d((d, d)),
        _const2d((1, d)),
        _const2d((d, d)),
        _const2d((1, d)),
        _const2d((d, 2 * d)),
        _const2d((1, 2 * d)),
        _const2d((1, 2 * d)),
        _const2d((1, 2 * d)),
        _const2d((2 * d, d)),
        _const2d((1, d)),
        _const2d((d, NS)),
        _const2d((1, NS)),
        pl.BlockSpec((SPB, d, d), lambda k: (k, 0, 0)),
        pl.BlockSpec((SPB, 1, d), lambda k: (k, 0, 0)),
        pl.BlockSpec((SPB, d, d), lambda k: (k, 0, 0)),
        pl.BlockSpec((SPB, 1, d), lambda k: (k, 0, 0)),
        pl.BlockSpec((2 * d, 2 * d), lambda k: (1, 0)),
        _const2d((1, 2 * d)),
        _const2d((1, 2 * d)),
        _const2d((1, 2 * d)),
        _const2d((2 * d, d)),
        _const2d((1, d)),
    ]
    out = pl.pallas_call(
        _body,
        grid=grid,
        in_specs=in_specs,
        out_specs=_const2d((1, d)),
        out_shape=jax.ShapeDtypeStruct((1, d), F32),
        scratch_shapes=[
            pltpu.VMEM((1, d), F32),
            pltpu.VMEM((1, d), F32),
            pltpu.VMEM((1, NS), F32),
        ],
        compiler_params=pltpu.CompilerParams(
            dimension_semantics=("arbitrary",),
            vmem_limit_bytes=67108864,
        ),
    )(x, concepts, Wq, row(bq), Wk, Wv, row(bv), Wo, row(bo),
      keW1, row(keb1), row(keg1), row(kebe1), keW2, row(keb2),
      selW, row(selb), skW1, skb1r, skW2, skb2r,
      fuW1, row(fub1), row(fug), row(fube), fuW2, row(fub2))
    return out


# speedup vs baseline: 1.2325x; 1.0644x over previous
import math

import jax
import jax.numpy as jnp
from jax.experimental import pallas as pl
from jax.experimental.pallas import tpu as pltpu

DIM = 768
H = 8
HD = DIM // H
NC = 1000
NS = 50
RING = 4
F32 = jnp.float32


def _body(x_ref, concepts_ref, wq_ref, bq_ref, wk_ref, wv_ref, bv_ref,
          wo_ref, bo_ref, keW1_ref, keb1_ref, keg1_ref, kebe1_ref,
          keW2_ref, keb2_ref, selW_ref, selb_ref,
          skW1_hbm, skb1_ref, skW2_hbm, skb2_ref,
          fuW1l_ref, fub1_ref, fug_ref, fube_ref, fuW2_ref, fub2_ref,
          out_ref, sem_ref, acc_ref, sc_ref, w1buf, w2buf, sems):
    k = pl.program_id(0)
    x = x_ref[...]

    def _issue(skill, slot):
        pltpu.make_async_copy(skW1_hbm.at[skill], w1buf.at[slot],
                              sems.at[slot, 0]).start()
        pltpu.make_async_copy(skW2_hbm.at[skill], w2buf.at[slot],
                              sems.at[slot, 1]).start()

    @pl.when(k == 0)
    def _prologue():
        for j in range(RING):
            _issue(j, j)

    @pl.when(k == 0)
    def _init():
        sl = jnp.dot(x, selW_ref[...], preferred_element_type=F32) + selb_ref[...]
        sl = sl - jnp.max(sl, axis=-1, keepdims=True)
        e = jnp.exp(sl)
        sc_ref[...] = e / jnp.sum(e, axis=-1, keepdims=True)

        q = jnp.dot(x, wq_ref[...], preferred_element_type=F32) + bq_ref[...]
        rows = jax.lax.broadcasted_iota(jnp.int32, (H, DIM), 0)
        cols = jax.lax.broadcasted_iota(jnp.int32, (H, DIM), 1)
        maskf = (cols // HD == rows).astype(F32)
        q8 = maskf * q
        t = jax.lax.dot_general(wk_ref[...], q8, (((1,), (1,)), ((), ())),
                                preferred_element_type=F32)
        logits = jnp.dot(concepts_ref[...], t,
                         preferred_element_type=F32) * (1.0 / math.sqrt(HD))
        m = jnp.max(logits, axis=0, keepdims=True)
        ew = jnp.exp(logits - m)
        w = ew / jnp.sum(ew, axis=0, keepdims=True)
        u = jax.lax.dot_general(w, concepts_ref[...], (((0,), (0,)), ((), ())),
                                preferred_element_type=F32)
        p = jnp.dot(u, wv_ref[...], preferred_element_type=F32)
        o = jnp.sum(p * maskf, axis=0, keepdims=True) + bv_ref[...]
        attended = jnp.dot(o, wo_ref[...], preferred_element_type=F32) + bo_ref[...]
        combined = x + attended
        y = jnp.dot(combined, keW1_ref[...], preferred_element_type=F32) + keb1_ref[...]
        mu = jnp.mean(y, axis=-1, keepdims=True)
        var = jnp.mean((y - mu) * (y - mu), axis=-1, keepdims=True)
        yn = (y - mu) / jnp.sqrt(var + 1e-5) * keg1_ref[...] + kebe1_ref[...]
        h1 = jnp.maximum(yn, 0.0)
        sem_ref[...] = jnp.dot(h1, keW2_ref[...], preferred_element_type=F32) + keb2_ref[...]

    slot = jax.lax.rem(k, RING)
    pltpu.make_async_copy(skW1_hbm.at[k], w1buf.at[slot], sems.at[slot, 0]).wait()
    pltpu.make_async_copy(skW2_hbm.at[k], w2buf.at[slot], sems.at[slot, 1]).wait()

    s_all = sc_ref[...]
    lane = jax.lax.broadcasted_iota(jnp.int32, (1, NS), 1)
    sk = jnp.sum(jnp.where(lane == k, s_all, 0.0))
    b1 = skb1_ref[pl.ds(k, 1), :]
    b2 = skb2_ref[pl.ds(k, 1), :]
    hk = jnp.maximum(
        jnp.dot(x, w1buf[slot], preferred_element_type=F32) + b1, 0.0)
    outk = jnp.dot(hk, w2buf[slot], preferred_element_type=F32) + b2
    contrib = sk * outk

    @pl.when(k == 0)
    def _first():
        acc_ref[...] = contrib

    @pl.when(k > 0)
    def _rest():
        acc_ref[...] = acc_ref[...] + contrib

    @pl.when(k + RING < NS)
    def _refill():
        _issue(k + RING, slot)

    @pl.when(k == NS - 1)
    def _fuse():
        cat = jnp.concatenate([sem_ref[...], acc_ref[...]], axis=-1)
        y = jnp.dot(cat, fuW1l_ref[...], preferred_element_type=F32) + fub1_ref[...]
        mu = jnp.mean(y, axis=-1, keepdims=True)
        var = jnp.mean((y - mu) * (y - mu), axis=-1, keepdims=True)
        yn = (y - mu) / jnp.sqrt(var + 1e-5) * fug_ref[...] + fube_ref[...]
        fh = jnp.maximum(yn, 0.0)
        out_ref[...] = jnp.dot(fh, fuW2_ref[...], preferred_element_type=F32) + fub2_ref[...]


def _const2d(shape):
    return pl.BlockSpec(shape, lambda k: (0, 0))


_HBM = pl.BlockSpec(memory_space=pltpu.MemorySpace.HBM)


@jax.jit
def kernel(x, concepts, Wq, bq, Wk, bk, Wv, bv, Wo, bo, keW1, keb1, keg1,
           kebe1, keW2, keb2, selW, selb, skW1, skb1, skW2, skb2,
           fuW1, fub1, fug, fube, fuW2, fub2):
    d = DIM
    row = lambda b: b.reshape(1, -1)
    grid = (NS,)
    in_specs = [
        _const2d((1, d)),
        _const2d((NC, d)),
        _const2d((d, d)),
        _const2d((1, d)),
        _const2d((d, d)),
        _const2d((d, d)),
        _const2d((1, d)),
        _const2d((d, d)),
        _const2d((1, d)),
        _const2d((d, 2 * d)),
        _const2d((1, 2 * d)),
        _const2d((1, 2 * d)),
        _const2d((1, 2 * d)),
        _const2d((2 * d, d)),
        _const2d((1, d)),
        _const2d((d, NS)),
        _const2d((1, NS)),
        _HBM,
        _const2d((NS, d)),
        _HBM,
        _const2d((NS, d)),
        pl.BlockSpec((2 * d, 2 * d), lambda k: (1, 0)),
        _const2d((1, 2 * d)),
        _const2d((1, 2 * d)),
        _const2d((1, 2 * d)),
        _const2d((2 * d, d)),
        _const2d((1, d)),
    ]
    out = pl.pallas_call(
        _body,
        grid=grid,
        in_specs=in_specs,
        out_specs=_const2d((1, d)),
        out_shape=jax.ShapeDtypeStruct((1, d), F32),
        scratch_shapes=[
            pltpu.VMEM((1, d), F32),
            pltpu.VMEM((1, d), F32),
            pltpu.VMEM((1, NS), F32),
            pltpu.VMEM((RING, d, d), F32),
            pltpu.VMEM((RING, d, d), F32),
            pltpu.SemaphoreType.DMA((RING, 2)),
        ],
        compiler_params=pltpu.CompilerParams(
            dimension_semantics=("arbitrary",),
            vmem_limit_bytes=67108864,
        ),
    )(x, concepts, Wq, row(bq), Wk, Wv, row(bv), Wo, row(bo),
      keW1, row(keb1), row(keg1), row(kebe1), keW2, row(keb2),
      selW, row(selb), skW1, skb1, skW2, skb2,
      fuW1, row(fub1), row(fug), row(fube), fuW2, row(fub2))
    return out
